# Initial kernel scaffold; baseline (speedup 1.0000x reference)
import jax
import jax.numpy as jnp
from jax.experimental import pallas as pl


def kernel(x, edge_index_inter, edge_index_intra, edge_index_aug, pos, batch, params):
    raise NotImplementedError("write your pallas kernel here")


# speedup vs baseline: 2.7454x; 2.7454x over previous
import functools

import jax
import jax.numpy as jnp
from jax import lax
from jax.experimental import pallas as pl
from jax.experimental.pallas import tpu as pltpu
from jax.experimental.pallas import tpu_sc as plsc

_NC = 2
_NS = 16
_CK = 128

_f32 = jnp.float32


def _sc_mesh():
    return plsc.VectorSubcoreMesh(core_axis_name="c", subcore_axis_name="s")


_SC_PARAMS = pltpu.CompilerParams(use_tc_tiling_on_sc=False)


@functools.lru_cache(maxsize=None)
def _make_rowgather(ne, nt, d):
    nchr = ne // _CK

    def body(table, idx2d, out, idxb, vbuf, sem):
        c = lax.axis_index("c")
        s = lax.axis_index("s")
        gid = c * _NS + s
        lo = gid * nchr // (_NC * _NS)
        hi = (gid + 1) * nchr // (_NC * _NS)

        def step(i, carry):
            pltpu.sync_copy(idx2d.at[i, :], idxb)
            pltpu.async_copy(table.at[idxb], vbuf, sem).wait()
            pltpu.sync_copy(vbuf, out.at[pl.ds(i * _CK, _CK), :])
            return carry

        lax.fori_loop(lo, hi, step, 0)

    return pl.kernel(
        body,
        out_type=jax.ShapeDtypeStruct((ne, d), _f32),
        mesh=_sc_mesh(),
        compiler_params=_SC_PARAMS,
        scratch_types=[
            pltpu.VMEM((_CK,), jnp.int32),
            pltpu.VMEM((_CK, d), _f32),
            pltpu.SemaphoreType.DMA,
        ],
    )


@functools.lru_cache(maxsize=None)
def _make_segsum(mode, ne, no, nt, d):
    nchr = ne // _CK
    rpt = no // _NS
    zr = min(625, rpt)
    nz = rpt // zr
    allcores = mode in ("gather", "edge32")
    ngroup = _NS if allcores else _NC * _NS

    def body(*refs):
        if mode == "gather":
            table, col2d, row2d, zeros, out, colb, rowb, vbuf, acc, sem = refs
        else:
            vals, col2d, zeros, out, colb, vbuf, acc, sem = refs
        c = lax.axis_index("c")
        s = lax.axis_index("s")
        for z in range(nz):
            pltpu.sync_copy(zeros, acc.at[pl.ds(s * rpt + z * zr, zr), :])
        plsc.subcore_barrier()

        gid = s if allcores else c * _NS + s
        lo = gid * nchr // ngroup
        hi = (gid + 1) * nchr // ngroup

        def step(i, carry):
            pltpu.sync_copy(col2d.at[i, :], colb)
            if mode == "gather":
                pltpu.sync_copy(row2d.at[i, :], rowb)
                off = c * nt
                for j in range(_CK // 16):
                    rowb[pl.ds(j * 16, 16)] = rowb[pl.ds(j * 16, 16)] + off
                pltpu.async_copy(table.at[rowb], vbuf, sem).wait()
            elif mode == "edge32":
                pltpu.sync_copy(vals.at[pl.ds(c * ne + i * _CK, _CK), :], vbuf)
            else:
                pltpu.sync_copy(vals.at[pl.ds(i * _CK, _CK), :], vbuf)
            pltpu.sync_copy(vbuf, acc.at[colb], add=True)
            return carry

        lax.fori_loop(lo, hi, step, 0)
        plsc.subcore_barrier()
        pltpu.sync_copy(acc.at[pl.ds(s * rpt, rpt), :],
                        out.at[pl.ds(c * no + s * rpt, rpt), :])

    scratch = [
        pltpu.VMEM((_CK,), jnp.int32),
        pltpu.VMEM((_CK, d), _f32),
        pltpu.VMEM_SHARED((no, d), _f32),
        pltpu.SemaphoreType.DMA,
    ]
    if mode == "gather":
        scratch.insert(1, pltpu.VMEM((_CK,), jnp.int32))

    return pl.kernel(
        body,
        out_type=jax.ShapeDtypeStruct((2 * no, d), _f32),
        mesh=_sc_mesh(),
        compiler_params=_SC_PARAMS,
        scratch_types=scratch,
    )


def _rowgather(table, idx):
    ne = idx.shape[0]
    fn = _make_rowgather(ne, table.shape[0], table.shape[1])
    return fn(table, idx.reshape(-1, _CK))


def _split2(t):
    return jnp.concatenate([t[:, :32], t[:, 32:]], axis=0)


def _unsplit2(t2, no):
    return jnp.concatenate([t2[:no], t2[no:]], axis=1)


def _segsum_gather(table64, row, col, no):
    ne = row.shape[0]
    t2 = _split2(table64)
    fn = _make_segsum("gather", ne, no, table64.shape[0], 32)
    z = jnp.zeros((min(625, no // _NS), 32), _f32)
    out = fn(t2, col.reshape(-1, _CK), row.reshape(-1, _CK), z)
    return _unsplit2(out, no)


def _segsum_edge32(vals2, col, no):
    ne = col.shape[0]
    fn = _make_segsum("edge32", ne, no, 0, 32)
    z = jnp.zeros((min(625, no // _NS), 32), _f32)
    out = fn(vals2, col.reshape(-1, _CK), z)
    return _unsplit2(out, no)


def _segsum_edge8(vals, col, no):
    ne = col.shape[0]
    fn = _make_segsum("edge8", ne, no, 0, 8)
    z = jnp.zeros((min(625, no // _NS), 8), _f32)
    out = fn(vals, col.reshape(-1, _CK), z)
    return jnp.stack([out[:no, 0], out[no:, 0]], axis=1)


def _blk(r):
    return 2000 if r % 2000 == 0 else r


def _activate(h, act):
    if act == "silu":
        return h * (1.0 / (1.0 + jnp.exp(-h)))
    if act == "leaky":
        return jnp.where(h >= 0, h, 0.01 * h)
    if act == "sigmoid":
        return 1.0 / (1.0 + jnp.exp(-h))
    return h


def _mm(terms, bias, act, degp=None):
    r = terms[0][0].shape[0]
    dout = terms[0][1].shape[1]
    br = _blk(r)
    grid = r // br
    nt = len(terms)

    def kern(*refs):
        o = refs[-1]
        acc = None
        for t in range(nt):
            a = refs[2 * t][...]
            if t == 0 and degp is not None:
                dp = refs[2 * nt + 1][...]
                a = a / jnp.maximum(dp[:, 0:1] + dp[:, 1:2], 1.0)
            p = jnp.dot(a, refs[2 * t + 1][...],
                        preferred_element_type=_f32)
            acc = p if acc is None else acc + p
        acc = acc + refs[2 * nt][...]
        o[...] = _activate(acc, act)

    in_specs = []
    args = []
    for a, w in terms:
        in_specs.append(pl.BlockSpec((br, a.shape[1]), lambda i: (i, 0)))
        in_specs.append(pl.BlockSpec(w.shape, lambda i: (0, 0)))
        args += [a, w]
    in_specs.append(pl.BlockSpec((1, dout), lambda i: (0, 0)))
    args.append(bias.reshape(1, dout))
    if degp is not None:
        in_specs.append(pl.BlockSpec((br, 2), lambda i: (i, 0)))
        args.append(degp)
    return pl.pallas_call(
        kern, grid=(grid,), in_specs=in_specs,
        out_specs=pl.BlockSpec((br, dout), lambda i: (i, 0)),
        out_shape=jax.ShapeDtypeStruct((r, dout), _f32),
    )(*args)


def _stats(h):
    r, d = h.shape
    br = _blk(r)

    def kern(h_ref, o_ref):
        b = h_ref[...]
        u = jnp.concatenate(
            [jnp.sum(b, 0, keepdims=True), jnp.sum(b * b, 0, keepdims=True)], 0)

        @pl.when(pl.program_id(0) == 0)
        def _():
            o_ref[...] = u

        @pl.when(pl.program_id(0) > 0)
        def _():
            o_ref[...] = o_ref[...] + u

    return pl.pallas_call(
        kern, grid=(r // br,),
        in_specs=[pl.BlockSpec((br, d), lambda i: (i, 0))],
        out_specs=pl.BlockSpec((2, d), lambda i: (0, 0)),
        out_shape=jax.ShapeDtypeStruct((2, d), _f32),
    )(h)


def _bn_apply(h, st, g, b):
    r, d = h.shape
    br = _blk(r)

    def kern(h_ref, s_ref, g_ref, b_ref, o_ref):
        m = s_ref[0:1, :] / r
        v = s_ref[1:2, :] / r - m * m
        o_ref[...] = ((h_ref[...] - m) * lax.rsqrt(v + 1e-5)
                      * g_ref[...] + b_ref[...])

    return pl.pallas_call(
        kern, grid=(r // br,),
        in_specs=[pl.BlockSpec((br, d), lambda i: (i, 0)),
                  pl.BlockSpec((2, d), lambda i: (0, 0)),
                  pl.BlockSpec((1, d), lambda i: (0, 0)),
                  pl.BlockSpec((1, d), lambda i: (0, 0))],
        out_specs=pl.BlockSpec((br, d), lambda i: (i, 0)),
        out_shape=jax.ShapeDtypeStruct((r, d), _f32),
    )(h, st, g.reshape(1, d), b.reshape(1, d))


def _batch_norm(h, g, b):
    return _bn_apply(h, _stats(h), g, b)


def _rbf_edge(pg0, pg1):
    e = pg0.shape[0]
    br = _blk(e)

    def kern(a_ref, b_ref, rbf_ref, d_ref):
        cd = a_ref[...] - b_ref[...]
        d = jnp.sqrt(jnp.sum(cd * cd, axis=1, keepdims=True) + 1e-12)
        mu = lax.broadcasted_iota(jnp.int32, (1, 16), 1).astype(_f32) * (
            6.0 / 15.0)
        z = (d - mu) / 0.375
        rbf_ref[...] = jnp.exp(-(z * z))
        d_ref[...] = d

    return pl.pallas_call(
        kern, grid=(e // br,),
        in_specs=[pl.BlockSpec((br, 16), lambda i: (i, 0)),
                  pl.BlockSpec((br, 16), lambda i: (i, 0))],
        out_specs=[pl.BlockSpec((br, 16), lambda i: (i, 0)),
                   pl.BlockSpec((br, 1), lambda i: (i, 0))],
        out_shape=[jax.ShapeDtypeStruct((e, 16), _f32),
                   jax.ShapeDtypeStruct((e, 1), _f32)],
    )(pg0, pg1)


def _gine_msg(ea_attr, w2, beu, g0, g1, xxg):
    e = ea_attr.shape[0]
    br = _blk(e)

    def kern(ea_ref, w_ref, b_ref, g0_ref, g1_ref, xg_ref, o_ref):
        u = (jnp.dot(ea_ref[...], w_ref[...], preferred_element_type=_f32)
             + g0_ref[...] + g1_ref[...] + b_ref[...])
        eu = _activate(u, "silu")
        m = jnp.maximum(xg_ref[...] + eu, 0.0)
        o_ref[0] = m[:, :32]
        o_ref[1] = m[:, 32:]

    out = pl.pallas_call(
        kern, grid=(e // br,),
        in_specs=[pl.BlockSpec((br, 64), lambda i: (i, 0)),
                  pl.BlockSpec((64, 64), lambda i: (0, 0)),
                  pl.BlockSpec((1, 64), lambda i: (0, 0)),
                  pl.BlockSpec((br, 64), lambda i: (i, 0)),
                  pl.BlockSpec((br, 64), lambda i: (i, 0)),
                  pl.BlockSpec((br, 64), lambda i: (i, 0))],
        out_specs=pl.BlockSpec((2, br, 32), lambda i: (0, i, 0)),
        out_shape=jax.ShapeDtypeStruct((2, e, 32), _f32),
    )(ea_attr, w2, beu.reshape(1, 64), g0, g1, xxg)
    return out.reshape(2 * e, 32)


def _appnp_msg(ew, g):
    e = ew.shape[0]
    br = _blk(e)

    def kern(w_ref, g_ref, o_ref):
        m = w_ref[...] * g_ref[...]
        o_ref[0] = m[:, :32]
        o_ref[1] = m[:, 32:]

    out = pl.pallas_call(
        kern, grid=(e // br,),
        in_specs=[pl.BlockSpec((br, 1), lambda i: (i, 0)),
                  pl.BlockSpec((br, 64), lambda i: (i, 0))],
        out_specs=pl.BlockSpec((2, br, 32), lambda i: (0, i, 0)),
        out_shape=jax.ShapeDtypeStruct((2, e, 32), _f32),
    )(ew, g)
    return out.reshape(2 * e, 32)


def _enc(x, we, be, wn, bn, degp):
    r = x.shape[0]
    br = _blk(r)

    def kern(x_ref, we_ref, be_ref, wn_ref, bn_ref, dp_ref, psc_ref,
             ydis_ref, xraw_ref):
        xb = x_ref[...]
        xl = _activate(jnp.dot(xb, we_ref[...], preferred_element_type=_f32)
                       + be_ref[...], "silu")
        nrm = jnp.sqrt(jnp.sum(xl * xl, axis=1, keepdims=True))
        psc = xl / jnp.maximum(nrm, 1e-12) * 1.8
        dp = dp_ref[...]
        dis = lax.rsqrt(dp[:, 0:1] + dp[:, 1:2] + 1.0)
        psc_ref[...] = psc
        ydis_ref[...] = dis * psc
        xraw_ref[...] = _activate(
            jnp.dot(xb, wn_ref[...], preferred_element_type=_f32)
            + bn_ref[...], "silu")

    return pl.pallas_call(
        kern, grid=(r // br,),
        in_specs=[pl.BlockSpec((br, 128), lambda i: (i, 0)),
                  pl.BlockSpec((128, 64), lambda i: (0, 0)),
                  pl.BlockSpec((1, 64), lambda i: (0, 0)),
                  pl.BlockSpec((128, 64), lambda i: (0, 0)),
                  pl.BlockSpec((1, 64), lambda i: (0, 0)),
                  pl.BlockSpec((br, 2), lambda i: (i, 0))],
        out_specs=[pl.BlockSpec((br, 64), lambda i: (i, 0))] * 3,
        out_shape=[jax.ShapeDtypeStruct((r, 64), _f32)] * 3,
    )(x, we, be.reshape(1, 64), wn, bn.reshape(1, 64), degp)


def _appnp_combine(sagg, psc, degp):
    r = sagg.shape[0]
    br = _blk(r)

    def kern(s_ref, p_ref, dp_ref, o_ref):
        dp = dp_ref[...]
        dis = lax.rsqrt(dp[:, 0:1] + dp[:, 1:2] + 1.0)
        p = p_ref[...]
        o_ref[...] = 0.9 * (dis * s_ref[...] + (dis * dis) * p) + 0.1 * p

    return pl.pallas_call(
        kern, grid=(r // br,),
        in_specs=[pl.BlockSpec((br, 64), lambda i: (i, 0)),
                  pl.BlockSpec((br, 64), lambda i: (i, 0)),
                  pl.BlockSpec((br, 2), lambda i: (i, 0))],
        out_specs=pl.BlockSpec((br, 64), lambda i: (i, 0)),
        out_shape=jax.ShapeDtypeStruct((r, 64), _f32),
    )(sagg, psc, degp)


def _head(pooled, xg, wo, bo):
    def kern(p_ref, x_ref, w1_ref, w2_ref, b_ref, o_ref):
        o_ref[...] = (jnp.dot(p_ref[...], w1_ref[...],
                              preferred_element_type=_f32)
                      + jnp.dot(x_ref[...], w2_ref[...],
                                preferred_element_type=_f32)
                      + b_ref[...])

    out = pl.pallas_call(
        kern,
        out_shape=jax.ShapeDtypeStruct((64, 1), _f32),
    )(pooled, xg, wo[:64], wo[64:], bo.reshape(1, 1))
    return out.reshape(-1)


def _gin_block(xx, xx2, agg, w, b, g, be):
    h = _mm([(agg, w), (xx, w)], b, "leaky")
    return _batch_norm(h, g, be)


def _dgnn(xx, row, col, degp, layers, n):
    h = xx
    for w, u, b in layers:
        agg = _segsum_gather(h, row, col, n)
        h = _mm([(agg, w), (h, u)], b, "silu", degp=degp)
    return h


def kernel(x, edge_index_inter, edge_index_intra, edge_index_aug, pos, batch,
           params):
    p = params
    n = x.shape[0]
    e = edge_index_inter.shape[1]
    g = 64

    ri, ci = edge_index_inter[0], edge_index_inter[1]
    ra, ca = edge_index_intra[0], edge_index_intra[1]
    rg, cg = edge_index_aug[0], edge_index_aug[1]

    pos16 = jnp.concatenate([pos, jnp.zeros((n, 13), _f32)], axis=1)
    pgi0 = _rowgather(pos16, ri)
    pgi1 = _rowgather(pos16, ci)
    pga0 = _rowgather(pos16, ra)
    pga1 = _rowgather(pos16, ca)
    rbf_i, ew = _rbf_edge(pgi0, pgi1)
    rbf_a, _ = _rbf_edge(pga0, pga1)
    wea, bea = p["edge_attr"]
    ea_i = _mm([(rbf_i, wea)], bea, "sigmoid")
    ea_a = _mm([(rbf_a, wea)], bea, "sigmoid")

    degp_app = _segsum_edge8(jnp.broadcast_to(ew, (e, 8)), ci, n)
    ones8 = jnp.ones((e, 8), _f32)
    degp_i = _segsum_edge8(ones8, ci, n)
    degp_a = _segsum_edge8(ones8, ca, n)

    we, be = p["enc_lin"]
    wn, bn0 = p["lin_node"]
    x_psc, ydis, x_raw = _enc(x, we, be, wn, bn0, degp_app)
    gy = _rowgather(ydis, ri)
    sagg = _segsum_edge32(_appnp_msg(ew, gy), ci, n)
    x_int0 = _appnp_combine(sagg, x_psc, degp_app)

    npad = 50048 - n
    bpad = jnp.concatenate([batch.astype(jnp.int32),
                            jnp.zeros((npad,), jnp.int32)])
    zpad = jnp.zeros((npad, 32), _f32)
    xi_vals = jnp.concatenate(
        [x_int0[:, :32], zpad, x_int0[:, 32:], zpad], axis=0)
    xg = _segsum_edge32(xi_vals, bpad, g)

    wm, bm, gm, bem = p["mlp_enc"]
    h = _mm([(x_int0, wm), (x_raw, wm)], bm, "leaky")
    xx = _batch_norm(h, gm, bem)

    weu, beu = p["edge_upd"]
    n0 = _mm([(xx, weu[:64])], jnp.zeros((64,), _f32), "none")
    n1 = _mm([(xx, weu[64:128])], jnp.zeros((64,), _f32), "none")
    w2 = weu[128:]

    def branch(row, col, ea_attr, gin_p, dgnn_layers, degp, lin_p):
        g0 = _rowgather(n0, row)
        g1 = _rowgather(n1, col)
        xxg = _rowgather(xx, row)
        msg = _gine_msg(ea_attr, w2, beu, g0, g1, xxg)
        agg = _segsum_edge32(msg, col, n)
        wgi, bgi, ggi, begi = gin_p
        x1 = _gin_block(xx, None, agg, wgi, bgi, ggi, begi)
        x2 = _dgnn(xx, row, col, degp, dgnn_layers, n)
        wl, bl = lin_p
        return _mm([(x1, wl[:64]), (x2, wl[64:])], bl, "silu")

    x_inter = branch(ri, ci, ea_i, p["gin1"], p["dgnn1"], degp_i, p["lin1"])
    x_intra = branch(ra, ca, ea_a, p["gin3"], p["dgnn3"], degp_a, p["lin3"])

    agg_m = _segsum_gather(xx, rg, cg, n)
    w4, b4, g4, be4 = p["gin4"]
    x_mask = _gin_block(xx, None, agg_m, w4, b4, g4, be4)

    wf1, bf1, gf1, bef1 = p["fc1"]
    h = _mm([(x_inter, wf1), (x_intra, wf1), (x_mask, wf1)], bf1, "leaky")
    h = _batch_norm(h, gf1, bef1)
    for name in ("fc2", "fc3"):
        w, b, gg, bb = p[name]
        h = _mm([(h, w)], b, "leaky")
        h = _batch_norm(h, gg, bb)
    w4f, b4f = p["fc4"]
    h = _mm([(h, w4f)], b4f, "none")

    h_vals = jnp.concatenate([h[:, :32], zpad, h[:, 32:], zpad], axis=0)
    pooled = _segsum_edge32(h_vals, bpad, g)

    wo, bo = p["lin_out"]
    return _head(pooled, xg, wo, bo)


# speedup vs baseline: 3.8307x; 1.3954x over previous
import functools

import jax
import jax.numpy as jnp
from jax import lax
from jax.experimental import pallas as pl
from jax.experimental.pallas import tpu as pltpu
from jax.experimental.pallas import tpu_sc as plsc

_NC = 2
_NS = 16
_CK = 128

_f32 = jnp.float32


def _sc_mesh():
    return plsc.VectorSubcoreMesh(core_axis_name="c", subcore_axis_name="s")


_SC_PARAMS = pltpu.CompilerParams(use_tc_tiling_on_sc=False)


def _sub_blocks(d):
    return {8: 16, 16: 16, 32: 16, 64: 8, 128: 4}[d]


@functools.lru_cache(maxsize=None)
def _make_rowgather(ne, nt, d):
    nchr = ne // _CK
    sb = _sub_blocks(d)

    def body(table, idx2d, out, idxb2, vbuf, sem):
        c = lax.axis_index("c")
        s = lax.axis_index("s")
        gid = c * _NS + s
        lo = gid * nchr // (_NC * _NS)
        hi = (gid + 1) * nchr // (_NC * _NS)
        nblk = (hi - lo) // sb

        def blk(b, carry):
            r0 = lo + b * sb
            pltpu.sync_copy(idx2d.at[pl.ds(r0, sb), :], idxb2)
            descs = [
                pltpu.async_copy(table.at[idxb2.at[j]],
                                 vbuf.at[pl.ds(j * _CK, _CK), :], sem)
                for j in range(sb)
            ]
            for de in descs:
                de.wait()
            pltpu.sync_copy(vbuf, out.at[pl.ds(r0 * _CK, sb * _CK), :])
            return carry

        lax.fori_loop(0, nblk, blk, 0)

        def tail(i, carry):
            pltpu.sync_copy(idx2d.at[i, :], idxb2.at[0])
            pltpu.async_copy(table.at[idxb2.at[0]],
                             vbuf.at[pl.ds(0, _CK), :], sem).wait()
            pltpu.sync_copy(vbuf.at[pl.ds(0, _CK), :],
                            out.at[pl.ds(i * _CK, _CK), :])
            return carry

        lax.fori_loop(lo + nblk * sb, hi, tail, 0)

    return pl.kernel(
        body,
        out_type=jax.ShapeDtypeStruct((ne, d), _f32),
        mesh=_sc_mesh(),
        compiler_params=_SC_PARAMS,
        scratch_types=[
            pltpu.VMEM((sb, _CK), jnp.int32),
            pltpu.VMEM((sb * _CK, d), _f32),
            pltpu.SemaphoreType.DMA,
        ],
    )


@functools.lru_cache(maxsize=None)
def _make_segsum(mode, ne, no, nt, d):
    nchr = ne // _CK
    rpt = no // _NS
    zr = min(625, rpt)
    nz = rpt // zr
    allcores = mode in ("gather", "edge32")
    ngroup = _NS if allcores else _NC * _NS

    free_words = 2_050_000 - no * d
    nidx = 2 if mode == "gather" else 1
    sb = max(1, min(16, free_words // (_NS * _CK * (d + nidx))))

    def body(*refs):
        if mode == "gather":
            (table, col2d, row2d, zeros, out,
             colb2, rowb2, vbuf, acc, sem, sem2) = refs
        else:
            vals, col2d, zeros, out, colb2, vbuf, acc, sem, sem2 = refs
        c = lax.axis_index("c")
        s = lax.axis_index("s")
        for z in range(nz):
            pltpu.sync_copy(zeros, acc.at[pl.ds(s * rpt + z * zr, zr), :])
        plsc.subcore_barrier()

        gid = s if allcores else c * _NS + s
        lo = gid * nchr // ngroup
        hi = (gid + 1) * nchr // ngroup
        nblk = (hi - lo) // sb
        off = c * nt

        def load_rows(r0, nsub):
            if mode == "gather":
                pltpu.sync_copy(row2d.at[pl.ds(r0, nsub), :],
                                rowb2.at[pl.ds(0, nsub), :])
                for j in range(nsub):
                    for k in range(_CK // 16):
                        rowb2[j, pl.ds(k * 16, 16)] = (
                            rowb2[j, pl.ds(k * 16, 16)] + off)
                descs = [
                    pltpu.async_copy(table.at[rowb2.at[j]],
                                     vbuf.at[pl.ds(j * _CK, _CK), :], sem)
                    for j in range(nsub)
                ]
                for de in descs:
                    de.wait()
            elif mode == "edge32":
                pltpu.sync_copy(
                    vals.at[pl.ds(c * ne + r0 * _CK, nsub * _CK), :],
                    vbuf.at[pl.ds(0, nsub * _CK), :])
            else:
                pltpu.sync_copy(vals.at[pl.ds(r0 * _CK, nsub * _CK), :],
                                vbuf.at[pl.ds(0, nsub * _CK), :])

        def blk(b, carry):
            r0 = lo + b * sb
            pltpu.sync_copy(col2d.at[pl.ds(r0, sb), :], colb2)
            load_rows(r0, sb)
            descs = [
                pltpu.async_copy(vbuf.at[pl.ds(j * _CK, _CK), :],
                                 acc.at[colb2.at[j]], sem2, add=True)
                for j in range(sb)
            ]
            for de in descs:
                de.wait()
            return carry

        lax.fori_loop(0, nblk, blk, 0)

        def tail(i, carry):
            pltpu.sync_copy(col2d.at[i, :], colb2.at[0])
            load_rows(i, 1)
            pltpu.async_copy(vbuf.at[pl.ds(0, _CK), :],
                             acc.at[colb2.at[0]], sem2, add=True).wait()
            return carry

        lax.fori_loop(lo + nblk * sb, hi, tail, 0)
        plsc.subcore_barrier()
        pltpu.sync_copy(acc.at[pl.ds(s * rpt, rpt), :],
                        out.at[pl.ds(c * no + s * rpt, rpt), :])

    scratch = [
        pltpu.VMEM((sb, _CK), jnp.int32),
        pltpu.VMEM((sb * _CK, d), _f32),
        pltpu.VMEM_SHARED((no, d), _f32),
        pltpu.SemaphoreType.DMA,
        pltpu.SemaphoreType.DMA,
    ]
    if mode == "gather":
        scratch.insert(1, pltpu.VMEM((sb, _CK), jnp.int32))

    return pl.kernel(
        body,
        out_type=jax.ShapeDtypeStruct((2 * no, d), _f32),
        mesh=_sc_mesh(),
        compiler_params=_SC_PARAMS,
        scratch_types=scratch,
    )


def _rowgather(table, idx):
    ne = idx.shape[0]
    fn = _make_rowgather(ne, table.shape[0], table.shape[1])
    return fn(table, idx.reshape(-1, _CK))


def _split2(t):
    return jnp.concatenate([t[:, :32], t[:, 32:]], axis=0)


def _unsplit2(t2, no):
    return jnp.concatenate([t2[:no], t2[no:]], axis=1)


def _segsum_gather(table64, row, col, no):
    ne = row.shape[0]
    t2 = _split2(table64)
    fn = _make_segsum("gather", ne, no, table64.shape[0], 32)
    z = jnp.zeros((min(625, no // _NS), 32), _f32)
    out = fn(t2, col.reshape(-1, _CK), row.reshape(-1, _CK), z)
    return _unsplit2(out, no)


def _segsum_edge32(vals2, col, no):
    ne = col.shape[0]
    fn = _make_segsum("edge32", ne, no, 0, 32)
    z = jnp.zeros((min(625, no // _NS), 32), _f32)
    out = fn(vals2, col.reshape(-1, _CK), z)
    return _unsplit2(out, no)


def _segsum_edge8(vals, col, no):
    ne = col.shape[0]
    fn = _make_segsum("edge8", ne, no, 0, 8)
    z = jnp.zeros((min(625, no // _NS), 8), _f32)
    out = fn(vals, col.reshape(-1, _CK), z)
    return jnp.stack([out[:no, 0], out[no:, 0]], axis=1)


def _blk(r):
    return 2000 if r % 2000 == 0 else r


def _activate(h, act):
    if act == "silu":
        return h * (1.0 / (1.0 + jnp.exp(-h)))
    if act == "leaky":
        return jnp.where(h >= 0, h, 0.01 * h)
    if act == "sigmoid":
        return 1.0 / (1.0 + jnp.exp(-h))
    return h


def _mm(groups, bias, act, degp=None):
    groups = [(list(als), w) for als, w in groups]
    r = groups[0][0][0].shape[0]
    dout = groups[0][1].shape[1]
    br = _blk(r)
    grid = r // br
    sizes = [len(als) for als, _ in groups]

    def kern(*refs):
        o = refs[-1]
        acc = None
        pos = 0
        nin = sum(sizes) + len(sizes)
        for gidx, ng in enumerate(sizes):
            a = refs[pos][...]
            for t in range(1, ng):
                a = a + refs[pos + t][...]
            if gidx == 0 and degp is not None:
                dp = refs[nin + 1][...]
                a = a / jnp.maximum(dp[:, 0:1] + dp[:, 1:2], 1.0)
            p = jnp.dot(a, refs[pos + ng][...],
                        preferred_element_type=_f32)
            acc = p if acc is None else acc + p
            pos += ng + 1
        acc = acc + refs[nin][...]
        o[...] = _activate(acc, act)

    in_specs = []
    args = []
    for als, w in groups:
        for a in als:
            in_specs.append(pl.BlockSpec((br, a.shape[1]), lambda i: (i, 0)))
            args.append(a)
        in_specs.append(pl.BlockSpec(w.shape, lambda i: (0, 0)))
        args.append(w)
    in_specs.append(pl.BlockSpec((1, dout), lambda i: (0, 0)))
    args.append(bias.reshape(1, dout))
    if degp is not None:
        in_specs.append(pl.BlockSpec((br, 2), lambda i: (i, 0)))
        args.append(degp)
    return pl.pallas_call(
        kern, grid=(grid,), in_specs=in_specs,
        out_specs=pl.BlockSpec((br, dout), lambda i: (i, 0)),
        out_shape=jax.ShapeDtypeStruct((r, dout), _f32),
    )(*args)


def _stats(h):
    r, d = h.shape
    br = _blk(r)

    def kern(h_ref, o_ref):
        b = h_ref[...]
        u = jnp.concatenate(
            [jnp.sum(b, 0, keepdims=True), jnp.sum(b * b, 0, keepdims=True)], 0)

        @pl.when(pl.program_id(0) == 0)
        def _():
            o_ref[...] = u

        @pl.when(pl.program_id(0) > 0)
        def _():
            o_ref[...] = o_ref[...] + u

    return pl.pallas_call(
        kern, grid=(r // br,),
        in_specs=[pl.BlockSpec((br, d), lambda i: (i, 0))],
        out_specs=pl.BlockSpec((2, d), lambda i: (0, 0)),
        out_shape=jax.ShapeDtypeStruct((2, d), _f32),
    )(h)


def _bn_apply(h, st, g, b):
    r, d = h.shape
    br = _blk(r)

    def kern(h_ref, s_ref, g_ref, b_ref, o_ref):
        m = s_ref[0:1, :] / r
        v = s_ref[1:2, :] / r - m * m
        o_ref[...] = ((h_ref[...] - m) * lax.rsqrt(v + 1e-5)
                      * g_ref[...] + b_ref[...])

    return pl.pallas_call(
        kern, grid=(r // br,),
        in_specs=[pl.BlockSpec((br, d), lambda i: (i, 0)),
                  pl.BlockSpec((2, d), lambda i: (0, 0)),
                  pl.BlockSpec((1, d), lambda i: (0, 0)),
                  pl.BlockSpec((1, d), lambda i: (0, 0))],
        out_specs=pl.BlockSpec((br, d), lambda i: (i, 0)),
        out_shape=jax.ShapeDtypeStruct((r, d), _f32),
    )(h, st, g.reshape(1, d), b.reshape(1, d))


def _batch_norm(h, g, b):
    return _bn_apply(h, _stats(h), g, b)


def _rbf_edge(pg0, pg1):
    e = pg0.shape[0]
    br = _blk(e)

    def kern(a_ref, b_ref, rbf_ref, d_ref):
        cd = a_ref[...] - b_ref[...]
        d = jnp.sqrt(jnp.sum(cd * cd, axis=1, keepdims=True) + 1e-12)
        mu = lax.broadcasted_iota(jnp.int32, (1, 16), 1).astype(_f32) * (
            6.0 / 15.0)
        z = (d - mu) / 0.375
        rbf_ref[...] = jnp.exp(-(z * z))
        d_ref[...] = d

    return pl.pallas_call(
        kern, grid=(e // br,),
        in_specs=[pl.BlockSpec((br, 16), lambda i: (i, 0)),
                  pl.BlockSpec((br, 16), lambda i: (i, 0))],
        out_specs=[pl.BlockSpec((br, 16), lambda i: (i, 0)),
                   pl.BlockSpec((br, 1), lambda i: (i, 0))],
        out_shape=[jax.ShapeDtypeStruct((e, 16), _f32),
                   jax.ShapeDtypeStruct((e, 1), _f32)],
    )(pg0, pg1)


def _gine_msg(ea_attr, w2, beu, gcat, g1):
    e = ea_attr.shape[0]
    br = _blk(e)

    def kern(ea_ref, w_ref, b_ref, gc_ref, g1_ref, o_ref):
        gc = gc_ref[...]
        u = (jnp.dot(ea_ref[...], w_ref[...], preferred_element_type=_f32)
             + gc[:, 64:] + g1_ref[...] + b_ref[...])
        eu = _activate(u, "silu")
        m = jnp.maximum(gc[:, :64] + eu, 0.0)
        o_ref[0] = m[:, :32]
        o_ref[1] = m[:, 32:]

    out = pl.pallas_call(
        kern, grid=(e // br,),
        in_specs=[pl.BlockSpec((br, 64), lambda i: (i, 0)),
                  pl.BlockSpec((64, 64), lambda i: (0, 0)),
                  pl.BlockSpec((1, 64), lambda i: (0, 0)),
                  pl.BlockSpec((br, 128), lambda i: (i, 0)),
                  pl.BlockSpec((br, 64), lambda i: (i, 0))],
        out_specs=pl.BlockSpec((2, br, 32), lambda i: (0, i, 0)),
        out_shape=jax.ShapeDtypeStruct((2, e, 32), _f32),
    )(ea_attr, w2, beu.reshape(1, 64), gcat, g1)
    return out.reshape(2 * e, 32)


def _appnp_msg(ew, g):
    e = ew.shape[0]
    br = _blk(e)

    def kern(w_ref, g_ref, o_ref):
        m = w_ref[...] * g_ref[...]
        o_ref[0] = m[:, :32]
        o_ref[1] = m[:, 32:]

    out = pl.pallas_call(
        kern, grid=(e // br,),
        in_specs=[pl.BlockSpec((br, 1), lambda i: (i, 0)),
                  pl.BlockSpec((br, 64), lambda i: (i, 0))],
        out_specs=pl.BlockSpec((2, br, 32), lambda i: (0, i, 0)),
        out_shape=jax.ShapeDtypeStruct((2, e, 32), _f32),
    )(ew, g)
    return out.reshape(2 * e, 32)


def _enc(x, we, be, wn, bn, degp):
    r = x.shape[0]
    br = _blk(r)

    def kern(x_ref, we_ref, be_ref, wn_ref, bn_ref, dp_ref, psc_ref,
             ydis_ref, xraw_ref):
        xb = x_ref[...]
        xl = _activate(jnp.dot(xb, we_ref[...], preferred_element_type=_f32)
                       + be_ref[...], "silu")
        nrm = jnp.sqrt(jnp.sum(xl * xl, axis=1, keepdims=True))
        psc = xl / jnp.maximum(nrm, 1e-12) * 1.8
        dp = dp_ref[...]
        dis = lax.rsqrt(dp[:, 0:1] + dp[:, 1:2] + 1.0)
        psc_ref[...] = psc
        ydis_ref[...] = dis * psc
        xraw_ref[...] = _activate(
            jnp.dot(xb, wn_ref[...], preferred_element_type=_f32)
            + bn_ref[...], "silu")

    return pl.pallas_call(
        kern, grid=(r // br,),
        in_specs=[pl.BlockSpec((br, 128), lambda i: (i, 0)),
                  pl.BlockSpec((128, 64), lambda i: (0, 0)),
                  pl.BlockSpec((1, 64), lambda i: (0, 0)),
                  pl.BlockSpec((128, 64), lambda i: (0, 0)),
                  pl.BlockSpec((1, 64), lambda i: (0, 0)),
                  pl.BlockSpec((br, 2), lambda i: (i, 0))],
        out_specs=[pl.BlockSpec((br, 64), lambda i: (i, 0))] * 3,
        out_shape=[jax.ShapeDtypeStruct((r, 64), _f32)] * 3,
    )(x, we, be.reshape(1, 64), wn, bn.reshape(1, 64), degp)


def _appnp_combine(sagg, psc, degp):
    r = sagg.shape[0]
    br = _blk(r)

    def kern(s_ref, p_ref, dp_ref, o_ref):
        dp = dp_ref[...]
        dis = lax.rsqrt(dp[:, 0:1] + dp[:, 1:2] + 1.0)
        p = p_ref[...]
        o_ref[...] = 0.9 * (dis * s_ref[...] + (dis * dis) * p) + 0.1 * p

    return pl.pallas_call(
        kern, grid=(r // br,),
        in_specs=[pl.BlockSpec((br, 64), lambda i: (i, 0)),
                  pl.BlockSpec((br, 64), lambda i: (i, 0)),
                  pl.BlockSpec((br, 2), lambda i: (i, 0))],
        out_specs=pl.BlockSpec((br, 64), lambda i: (i, 0)),
        out_shape=jax.ShapeDtypeStruct((r, 64), _f32),
    )(sagg, psc, degp)


def _head(pooled, xg, wo, bo):
    def kern(p_ref, x_ref, w1_ref, w2_ref, b_ref, o_ref):
        o_ref[...] = (jnp.dot(p_ref[...], w1_ref[...],
                              preferred_element_type=_f32)
                      + jnp.dot(x_ref[...], w2_ref[...],
                                preferred_element_type=_f32)
                      + b_ref[...])

    out = pl.pallas_call(
        kern,
        out_shape=jax.ShapeDtypeStruct((64, 1), _f32),
    )(pooled, xg, wo[:64], wo[64:], bo.reshape(1, 1))
    return out.reshape(-1)


def _gin_block(xx, xx2, agg, w, b, g, be):
    h = _mm([([agg, xx], w)], b, "leaky")
    return _batch_norm(h, g, be)


def _dgnn(xx, row, col, degp, layers, n):
    h = xx
    for w, u, b in layers:
        agg = _segsum_gather(h, row, col, n)
        h = _mm([([agg], w), ([h], u)], b, "silu", degp=degp)
    return h


def kernel(x, edge_index_inter, edge_index_intra, edge_index_aug, pos, batch,
           params):
    p = params
    n = x.shape[0]
    e = edge_index_inter.shape[1]
    g = 64

    ri, ci = edge_index_inter[0], edge_index_inter[1]
    ra, ca = edge_index_intra[0], edge_index_intra[1]
    rg, cg = edge_index_aug[0], edge_index_aug[1]

    pos16 = jnp.concatenate([pos, jnp.zeros((n, 13), _f32)], axis=1)
    pgi0 = _rowgather(pos16, ri)
    pgi1 = _rowgather(pos16, ci)
    pga0 = _rowgather(pos16, ra)
    pga1 = _rowgather(pos16, ca)
    rbf_i, ew = _rbf_edge(pgi0, pgi1)
    rbf_a, _ = _rbf_edge(pga0, pga1)
    wea, bea = p["edge_attr"]
    ea_i = _mm([([rbf_i], wea)], bea, "sigmoid")
    ea_a = _mm([([rbf_a], wea)], bea, "sigmoid")

    degp_app = _segsum_edge8(jnp.broadcast_to(ew, (e, 8)), ci, n)
    ones8 = jnp.ones((e, 8), _f32)
    degp_i = _segsum_edge8(ones8, ci, n)
    degp_a = _segsum_edge8(ones8, ca, n)

    we, be = p["enc_lin"]
    wn, bn0 = p["lin_node"]
    x_psc, ydis, x_raw = _enc(x, we, be, wn, bn0, degp_app)
    gy = _rowgather(ydis, ri)
    sagg = _segsum_edge32(_appnp_msg(ew, gy), ci, n)
    x_int0 = _appnp_combine(sagg, x_psc, degp_app)

    npad = 50048 - n
    bpad = jnp.concatenate([batch.astype(jnp.int32),
                            jnp.zeros((npad,), jnp.int32)])
    zpad = jnp.zeros((npad, 32), _f32)
    xi_vals = jnp.concatenate(
        [x_int0[:, :32], zpad, x_int0[:, 32:], zpad], axis=0)
    xg = _segsum_edge32(xi_vals, bpad, g)

    wm, bm, gm, bem = p["mlp_enc"]
    h = _mm([([x_int0, x_raw], wm)], bm, "leaky")
    xx = _batch_norm(h, gm, bem)

    weu, beu = p["edge_upd"]
    n0 = _mm([([xx], weu[:64])], jnp.zeros((64,), _f32), "none")
    n1 = _mm([([xx], weu[64:128])], jnp.zeros((64,), _f32), "none")
    w2 = weu[128:]
    xxn0 = jnp.concatenate([xx, n0], axis=1)

    def branch(row, col, ea_attr, gin_p, dgnn_layers, degp, lin_p):
        gcat = _rowgather(xxn0, row)
        g1 = _rowgather(n1, col)
        msg = _gine_msg(ea_attr, w2, beu, gcat, g1)
        agg = _segsum_edge32(msg, col, n)
        wgi, bgi, ggi, begi = gin_p
        x1 = _gin_block(xx, None, agg, wgi, bgi, ggi, begi)
        x2 = _dgnn(xx, row, col, degp, dgnn_layers, n)
        wl, bl = lin_p
        return _mm([([x1], wl[:64]), ([x2], wl[64:])], bl, "silu")

    x_inter = branch(ri, ci, ea_i, p["gin1"], p["dgnn1"], degp_i, p["lin1"])
    x_intra = branch(ra, ca, ea_a, p["gin3"], p["dgnn3"], degp_a, p["lin3"])

    agg_m = _segsum_gather(xx, rg, cg, n)
    w4, b4, g4, be4 = p["gin4"]
    x_mask = _gin_block(xx, None, agg_m, w4, b4, g4, be4)

    wf1, bf1, gf1, bef1 = p["fc1"]
    h = _mm([([x_inter, x_intra, x_mask], wf1)], bf1, "leaky")
    h = _batch_norm(h, gf1, bef1)
    for name in ("fc2", "fc3"):
        w, b, gg, bb = p[name]
        h = _mm([([h], w)], b, "leaky")
        h = _batch_norm(h, gg, bb)
    w4f, b4f = p["fc4"]
    h = _mm([([h], w4f)], b4f, "none")

    h_vals = jnp.concatenate([h[:, :32], zpad, h[:, 32:], zpad], axis=0)
    pooled = _segsum_edge32(h_vals, bpad, g)

    wo, bo = p["lin_out"]
    return _head(pooled, xg, wo, bo)
